# baseline (speedup 1.0000x reference)
import functools

import jax
import jax.numpy as jnp
from jax import lax
from jax.experimental import pallas as pl
from jax.experimental.pallas import tpu as pltpu
from jax.experimental.pallas import tpu_sc as plsc

ENTITY_N = 100000
RELATION_N = 1000
D = 128
B = 4096
NW = 32
BPW = B // NW


def _lane_allsum(v):
    lanes = lax.iota(jnp.int32, 16)
    dnums = lax.GatherDimensionNumbers(
        offset_dims=(), collapsed_slice_dims=(0,), start_index_map=(0,))
    for sh in (8, 4, 2, 1):
        perm = jnp.bitwise_xor(lanes, sh)
        v = v + lax.gather(
            v, perm[:, None], dnums, slice_sizes=(1,),
            mode=lax.GatherScatterMode.PROMISE_IN_BOUNDS)
    return v


def _inv_norm(sv):
    iv = plsc.bitcast(sv, jnp.int32)
    iv = jnp.int32(0x5F3759DF) - lax.shift_right_logical(iv, 1)
    y = plsc.bitcast(iv, jnp.float32)
    y = y * (1.5 - 0.5 * sv * y * y)
    y = y * (1.5 - 0.5 * sv * y * y)
    y = y * (1.5 - 0.5 * sv * y * y)
    n = sv * y
    return 1.0 / jnp.maximum(n, 1e-12)


def _make_sc_kernel():
    mesh = plsc.VectorSubcoreMesh(core_axis_name="c", subcore_axis_name="s")

    @functools.partial(
        pl.kernel,
        out_type=jax.ShapeDtypeStruct((3 * B, D), jnp.float32),
        mesh=mesh,
        compiler_params=pltpu.CompilerParams(needs_layout_passes=False),
        scratch_types=[
            pltpu.VMEM((BPW,), jnp.int32),
            pltpu.VMEM((BPW,), jnp.int32),
            pltpu.VMEM((BPW,), jnp.int32),
            pltpu.VMEM((BPW, D), jnp.float32),
            pltpu.VMEM((BPW, D), jnp.float32),
            pltpu.VMEM((BPW, D), jnp.float32),
            pltpu.VMEM((3 * BPW, D), jnp.float32),
            pltpu.SemaphoreType.DMA,
        ],
    )
    def body(idx_h, idx_r, idx_t, entity, relation, out,
             ih_v, ir_v, it_v, buf_h, buf_r, buf_t, obuf, sem):
        wid = lax.axis_index("s") * 2 + lax.axis_index("c")
        b0 = wid * BPW
        pltpu.sync_copy(idx_h.at[pl.ds(b0, BPW)], ih_v)
        pltpu.sync_copy(idx_r.at[pl.ds(b0, BPW)], ir_v)
        pltpu.sync_copy(idx_t.at[pl.ds(b0, BPW)], it_v)
        ch = pltpu.async_copy(entity.at[ih_v], buf_h, sem)
        cr = pltpu.async_copy(relation.at[ir_v], buf_r, sem)
        ct = pltpu.async_copy(entity.at[it_v], buf_t, sem)
        ch.wait()
        cr.wait()
        ct.wait()

        def row_body(i, _):
            for c, buf in ((0, buf_h), (1, buf_r), (2, buf_t)):
                acc = jnp.zeros((16,), jnp.float32)
                for k in range(D // 16):
                    v = buf[i, pl.ds(16 * k, 16)]
                    acc = acc + v * v
                inv = _inv_norm(_lane_allsum(acc))
                for k in range(D // 16):
                    obuf[3 * i + c, pl.ds(16 * k, 16)] = (
                        buf[i, pl.ds(16 * k, 16)] * inv)
            return 0

        lax.fori_loop(0, BPW, row_body, 0)
        pltpu.sync_copy(obuf, out.at[pl.ds(3 * b0, 3 * BPW)])

    return body


_sc_kernel = _make_sc_kernel()


def kernel(sample, entity_emb, relation_emb, loss_emb):
    del loss_emb
    idx_h = sample[:, 0].astype(jnp.int32)
    idx_r = sample[:, 1].astype(jnp.int32)
    idx_t = sample[:, 2].astype(jnp.int32)
    flat = _sc_kernel(idx_h, idx_r, idx_t, entity_emb, relation_emb)
    return flat.reshape(B, 3, D)
